# baseline (speedup 1.0000x reference)
import functools

import jax
import jax.numpy as jnp
from jax import lax
from jax.experimental import pallas as pl
from jax.experimental.pallas import tpu as pltpu
from jax.experimental.pallas import tpu_sc as plsc

N_NODES = 10000
N_EDGES = 160000
D = 256
NQ = 4
FW = D // NQ
NP = NQ + 1

NPAD = 10240
ROWS_PER_SUB = NPAD // 16
EPAD = 163840
CW = 128
K = EPAD // CW // 32
NBUF = 4

_mesh = plsc.VectorSubcoreMesh(core_axis_name="c", subcore_axis_name="s")


@functools.partial(
    pl.kernel,
    mesh=_mesh,
    compiler_params=pltpu.CompilerParams(use_tc_tiling_on_sc=False),
    out_type=jax.ShapeDtypeStruct((2, NP, NPAD, FW), jnp.float32),
    scratch_types=[
        pltpu.VMEM((K, CW), jnp.int32),
        pltpu.VMEM((K, CW), jnp.int32),
        pltpu.VMEM((K, CW), jnp.int32),
        [pltpu.VMEM((CW, FW), jnp.float32) for _ in range(NBUF)],
        pltpu.VMEM((CW, FW), jnp.float32),
        pltpu.VMEM((CW, FW), jnp.float32),
        pltpu.VMEM_SHARED((NPAD, FW), jnp.float32),
        [pltpu.SemaphoreType.DMA for _ in range(NBUF)],
        [pltpu.SemaphoreType.DMA for _ in range(NBUF)],
        pltpu.SemaphoreType.DMA,
    ],
)
def _sc_aggregate(tbl, src_a, dst_a, psum,
                  idx_r, idx_s, idx_d, rows, zrows, ones_v, acc_sh,
                  sg, ss, swb):
    c = lax.axis_index("c")
    s = lax.axis_index("s")
    base = s * ROWS_PER_SUB
    wid = c * 16 + s

    def _fill_const(i, _):
        for l in range(FW // 16):
            zrows[i, pl.ds(l * 16, 16)] = jnp.zeros((16,), jnp.float32)
            ones_v[i, pl.ds(l * 16, 16)] = jnp.ones((16,), jnp.float32)
        return 0

    lax.fori_loop(0, CW, _fill_const, 0, unroll=False)

    def _zero_stripe():
        def _z(t, _):
            pltpu.sync_copy(zrows, acc_sh.at[pl.ds(base + t * CW, CW)])
            return 0
        lax.fori_loop(0, ROWS_PER_SUB // CW, _z, 0, unroll=False)

    def _writeback(p):
        for t in range(ROWS_PER_SUB // CW):
            b = t % 2
            if t >= 2:
                pltpu.make_async_copy(
                    rows[b], psum.at[c, p, pl.ds(base, CW)], swb).wait()
            pltpu.sync_copy(acc_sh.at[pl.ds(base + t * CW, CW)], rows[b])
            pltpu.async_copy(
                rows[b], psum.at[c, p, pl.ds(base + t * CW, CW)], swb)
        for t in range(2):
            pltpu.make_async_copy(
                rows[t], psum.at[c, p, pl.ds(base, CW)], swb).wait()

    pltpu.sync_copy(src_a.at[wid], idx_r)
    pltpu.sync_copy(dst_a.at[wid], idx_d)

    _zero_stripe()
    plsc.subcore_barrier()

    def _quarter(q, _):
        def _mkidx(i, __):
            for l in range(CW // 16):
                idx_s[i, pl.ds(l * 16, 16)] = (
                    idx_r[i, pl.ds(l * 16, 16)] * NQ + q)
            return 0

        lax.fori_loop(0, K, _mkidx, 0, unroll=False)

        for b in range(NBUF):
            pltpu.async_copy(tbl.at[idx_s.at[b]], rows[b], sg[b])

        def _ring_block(t, __):
            for b in range(NBUF):
                j = t * NBUF + b
                pltpu.make_async_copy(
                    tbl.at[idx_s.at[0]], rows[b], sg[b]).wait()
                pltpu.async_copy(
                    rows[b], acc_sh.at[idx_d.at[j]], ss[b], add=True)
                pltpu.make_async_copy(
                    rows[b], acc_sh.at[idx_d.at[0]], ss[b]).wait()

                @pl.when(t < K // NBUF - 1)
                def _():
                    pltpu.async_copy(
                        tbl.at[idx_s.at[j + NBUF]], rows[b], sg[b])
            return 0

        lax.fori_loop(0, K // NBUF, _ring_block, 0, unroll=False)
        plsc.subcore_barrier()
        _writeback(q)
        _zero_stripe()
        plsc.subcore_barrier()
        return 0

    lax.fori_loop(0, NQ, _quarter, 0, unroll=False)

    def _deg_block(t, _):
        for b in range(NBUF):
            j = t * NBUF + b
            pltpu.async_copy(ones_v, acc_sh.at[idx_d.at[j]], ss[b], add=True)
        for b in range(NBUF):
            pltpu.make_async_copy(
                ones_v, acc_sh.at[idx_d.at[0]], ss[b]).wait()
        return 0

    lax.fori_loop(0, K // NBUF, _deg_block, 0, unroll=False)
    plsc.subcore_barrier()
    _writeback(NQ)


BLK = 2000


def _tc_self_body(x_ref, wst, b_ref, o_ref):
    o_ref[...] = (
        jnp.dot(x_ref[...], wst[...], preferred_element_type=jnp.float32)
        + b_ref[...]
    )


def _tc_self(x, wst, b2d):
    return pl.pallas_call(
        _tc_self_body,
        grid=(N_NODES // BLK,),
        in_specs=[
            pl.BlockSpec((BLK, D), lambda i: (i, 0)),
            pl.BlockSpec((D, D), lambda i: (0, 0)),
            pl.BlockSpec((1, D), lambda i: (0, 0)),
        ],
        out_specs=pl.BlockSpec((BLK, D), lambda i: (i, 0)),
        out_shape=jax.ShapeDtypeStruct((N_NODES, D), jnp.float32),
    )(x, wst, b2d)


def _tc_tail_body(self_ref, ps, wnt, o_ref):
    deg = jnp.maximum(ps[0, NQ, :, 0:1] + ps[1, NQ, :, 0:1], 1.0)
    hn = jnp.concatenate(
        [ps[0, q] + ps[1, q] for q in range(NQ)], axis=1) / deg
    o_ref[...] = self_ref[...] + jnp.dot(
        hn, wnt[...], preferred_element_type=jnp.float32)


def _tc_tail(self_part, psum, wnt):
    return pl.pallas_call(
        _tc_tail_body,
        grid=(N_NODES // BLK,),
        in_specs=[
            pl.BlockSpec((BLK, D), lambda i: (i, 0)),
            pl.BlockSpec((2, NP, BLK, FW), lambda i: (0, 0, i, 0)),
            pl.BlockSpec((D, D), lambda i: (0, 0)),
        ],
        out_specs=pl.BlockSpec((BLK, D), lambda i: (i, 0)),
        out_shape=jax.ShapeDtypeStruct((N_NODES, D), jnp.float32),
    )(self_part, psum, wnt)


def kernel(x, edge_index, W_self, W_neigh, b):
    x = x.astype(jnp.float32)
    src = edge_index[0].astype(jnp.int32)
    dst = edge_index[1].astype(jnp.int32)

    tbl = x.reshape(N_NODES * NQ, FW)

    npad_e = EPAD - N_EDGES
    pad_src = (jnp.arange(npad_e, dtype=jnp.int32) * 41) % N_NODES
    pad_dst = N_NODES + (
        jnp.arange(npad_e, dtype=jnp.int32) % (NPAD - N_NODES))
    src_a = jnp.concatenate([src, pad_src]).reshape(32, K, CW)
    dst_a = jnp.concatenate([dst, pad_dst]).reshape(32, K, CW)

    self_part = _tc_self(x, W_self.T, b.reshape(1, D))
    psum = _sc_aggregate(tbl, src_a, dst_a)

    return _tc_tail(self_part, psum, W_neigh.T)


# speedup vs baseline: 1.0623x; 1.0623x over previous
import functools

import jax
import jax.numpy as jnp
from jax import lax
from jax.experimental import pallas as pl
from jax.experimental.pallas import tpu as pltpu
from jax.experimental.pallas import tpu_sc as plsc

N_NODES = 10000
N_EDGES = 160000
D = 256
NQ = 4
FW = D // NQ

NPAD = 10240
ROWS_PER_SUB = NPAD // 16
EPAD = 163840
CW = 128
K = EPAD // CW // 32
NBUF = 4

_mesh = plsc.VectorSubcoreMesh(core_axis_name="c", subcore_axis_name="s")


@functools.partial(
    pl.kernel,
    mesh=_mesh,
    compiler_params=pltpu.CompilerParams(use_tc_tiling_on_sc=False),
    out_type=[
        jax.ShapeDtypeStruct((2, NQ, NPAD, FW), jnp.float32),
        jax.ShapeDtypeStruct((2, NPAD, 16), jnp.float32),
    ],
    scratch_types=[
        pltpu.VMEM((K, CW), jnp.int32),
        pltpu.VMEM((K, CW), jnp.int32),
        pltpu.VMEM((K, CW), jnp.int32),
        [pltpu.VMEM((CW, FW), jnp.float32) for _ in range(NBUF)],
        pltpu.VMEM((CW, FW), jnp.float32),
        pltpu.VMEM((CW, 16), jnp.float32),
        pltpu.VMEM((ROWS_PER_SUB, 16), jnp.float32),
        pltpu.VMEM_SHARED((NPAD, FW), jnp.float32),
        pltpu.VMEM_SHARED((NPAD, 16), jnp.float32),
        [pltpu.SemaphoreType.DMA for _ in range(NBUF)],
        [pltpu.SemaphoreType.DMA for _ in range(NBUF)],
        [pltpu.SemaphoreType.DMA for _ in range(NBUF)],
        pltpu.SemaphoreType.DMA,
    ],
)
def _sc_aggregate(tbl, src_a, dst_a, psum, pdeg,
                  idx_r, idx_s, idx_d, rows, zrows, ones_v, dbuf,
                  acc_sh, deg_sh, sg, ss, sd, swb):
    c = lax.axis_index("c")
    s = lax.axis_index("s")
    base = s * ROWS_PER_SUB
    wid = c * 16 + s

    def _fill_const(i, _):
        for l in range(FW // 16):
            zrows[i, pl.ds(l * 16, 16)] = jnp.zeros((16,), jnp.float32)
        ones_v[i, :] = jnp.ones((16,), jnp.float32)
        return 0

    lax.fori_loop(0, CW, _fill_const, 0, unroll=False)

    def _fill_dbuf(i, _):
        dbuf[i, :] = jnp.zeros((16,), jnp.float32)
        return 0

    lax.fori_loop(0, ROWS_PER_SUB, _fill_dbuf, 0, unroll=False)

    def _zero_stripe():
        def _z(t, _):
            pltpu.sync_copy(zrows, acc_sh.at[pl.ds(base + t * CW, CW)])
            return 0
        lax.fori_loop(0, ROWS_PER_SUB // CW, _z, 0, unroll=False)

    def _writeback(p):
        for t in range(ROWS_PER_SUB // CW):
            b = t % 2
            if t >= 2:
                pltpu.make_async_copy(
                    rows[b], psum.at[c, p, pl.ds(base, CW)], swb).wait()
            pltpu.sync_copy(acc_sh.at[pl.ds(base + t * CW, CW)], rows[b])
            pltpu.async_copy(
                rows[b], psum.at[c, p, pl.ds(base + t * CW, CW)], swb)
        for t in range(2):
            pltpu.make_async_copy(
                rows[t], psum.at[c, p, pl.ds(base, CW)], swb).wait()

    def _mkidx(q):
        def _m(i, __):
            for l in range(CW // 16):
                idx_s[i, pl.ds(l * 16, 16)] = (
                    idx_r[i, pl.ds(l * 16, 16)] * NQ + q)
            return 0
        lax.fori_loop(0, K, _m, 0, unroll=False)

    def _ring(with_deg):
        for b in range(NBUF):
            pltpu.async_copy(tbl.at[idx_s.at[b]], rows[b], sg[b])

        def _ring_block(t, __):
            for b in range(NBUF):
                j = t * NBUF + b
                pltpu.make_async_copy(
                    tbl.at[idx_s.at[0]], rows[b], sg[b]).wait()
                pltpu.async_copy(
                    rows[b], acc_sh.at[idx_d.at[j]], ss[b], add=True)
                if with_deg:
                    pltpu.async_copy(
                        ones_v, deg_sh.at[idx_d.at[j]], sd[b], add=True)
                pltpu.make_async_copy(
                    rows[b], acc_sh.at[idx_d.at[0]], ss[b]).wait()
                if with_deg:
                    pltpu.make_async_copy(
                        ones_v, deg_sh.at[idx_d.at[0]], sd[b]).wait()

                @pl.when(t < K // NBUF - 1)
                def _():
                    pltpu.async_copy(
                        tbl.at[idx_s.at[j + NBUF]], rows[b], sg[b])
            return 0

        lax.fori_loop(0, K // NBUF, _ring_block, 0, unroll=False)

    pltpu.sync_copy(src_a.at[wid], idx_r)
    pltpu.sync_copy(dst_a.at[wid], idx_d)

    _zero_stripe()
    pltpu.sync_copy(dbuf, deg_sh.at[pl.ds(base, ROWS_PER_SUB)])
    plsc.subcore_barrier()

    _mkidx(0)
    _ring(with_deg=True)
    plsc.subcore_barrier()
    _writeback(0)
    pltpu.sync_copy(deg_sh.at[pl.ds(base, ROWS_PER_SUB)], dbuf)
    pltpu.sync_copy(dbuf, pdeg.at[c, pl.ds(base, ROWS_PER_SUB)])
    _zero_stripe()
    plsc.subcore_barrier()

    def _quarter(q, _):
        _mkidx(q)
        _ring(with_deg=False)
        plsc.subcore_barrier()
        _writeback(q)

        @pl.when(q < NQ - 1)
        def _():
            _zero_stripe()
        plsc.subcore_barrier()
        return 0

    lax.fori_loop(1, NQ, _quarter, 0, unroll=False)


BLK = 2000


def _tc_self_body(x_ref, wst, b_ref, o_ref):
    o_ref[...] = (
        jnp.dot(x_ref[...], wst[...], preferred_element_type=jnp.float32)
        + b_ref[...]
    )


def _tc_self(x, wst, b2d):
    return pl.pallas_call(
        _tc_self_body,
        grid=(N_NODES // BLK,),
        in_specs=[
            pl.BlockSpec((BLK, D), lambda i: (i, 0)),
            pl.BlockSpec((D, D), lambda i: (0, 0)),
            pl.BlockSpec((1, D), lambda i: (0, 0)),
        ],
        out_specs=pl.BlockSpec((BLK, D), lambda i: (i, 0)),
        out_shape=jax.ShapeDtypeStruct((N_NODES, D), jnp.float32),
    )(x, wst, b2d)


def _tc_tail_body(self_ref, ps, dg, wnt, o_ref):
    deg = jnp.maximum(dg[0, :, 0:1] + dg[1, :, 0:1], 1.0)
    hn = jnp.concatenate(
        [ps[0, q] + ps[1, q] for q in range(NQ)], axis=1) / deg
    o_ref[...] = self_ref[...] + jnp.dot(
        hn, wnt[...], preferred_element_type=jnp.float32)


def _tc_tail(self_part, psum, pdeg, wnt):
    return pl.pallas_call(
        _tc_tail_body,
        grid=(N_NODES // BLK,),
        in_specs=[
            pl.BlockSpec((BLK, D), lambda i: (i, 0)),
            pl.BlockSpec((2, NQ, BLK, FW), lambda i: (0, 0, i, 0)),
            pl.BlockSpec((2, BLK, 16), lambda i: (0, i, 0)),
            pl.BlockSpec((D, D), lambda i: (0, 0)),
        ],
        out_specs=pl.BlockSpec((BLK, D), lambda i: (i, 0)),
        out_shape=jax.ShapeDtypeStruct((N_NODES, D), jnp.float32),
    )(self_part, psum, pdeg, wnt)


def kernel(x, edge_index, W_self, W_neigh, b):
    x = x.astype(jnp.float32)
    src = edge_index[0].astype(jnp.int32)
    dst = edge_index[1].astype(jnp.int32)

    tbl = x.reshape(N_NODES * NQ, FW)

    npad_e = EPAD - N_EDGES
    pad_src = (jnp.arange(npad_e, dtype=jnp.int32) * 41) % N_NODES
    pad_dst = N_NODES + (
        jnp.arange(npad_e, dtype=jnp.int32) % (NPAD - N_NODES))
    src_a = jnp.concatenate([src, pad_src]).reshape(32, K, CW)
    dst_a = jnp.concatenate([dst, pad_dst]).reshape(32, K, CW)

    psum, pdeg = _sc_aggregate(tbl, src_a, dst_a)
    self_part = _tc_self(x, W_self.T, b.reshape(1, D))

    return _tc_tail(self_part, psum, pdeg, W_neigh.T)
